# Initial kernel scaffold; baseline (speedup 1.0000x reference)
import jax
import jax.numpy as jnp
from jax.experimental import pallas as pl


def kernel(node_features, edge_index, edge_attrs, edge_embedding, W1, Wfc1, bfc1, Wfc2, W2, Wsc):
    raise NotImplementedError("write your pallas kernel here")


# speedup vs baseline: 1.6049x; 1.6049x over previous
import functools

import jax
import jax.numpy as jnp
import numpy as np
from jax import lax
from jax.experimental import pallas as pl
from jax.experimental.pallas import tpu as pltpu
from jax.experimental.pallas import tpu_sc as plsc

N_NODES = 10000
D = 128
E = 320000
D_EDGE = 16
FC_HIDDEN = 64
AVG_NUM_NEIGHBORS = 32.0
LOG2 = float(np.log(2.0))

NCORES = 2
NSUB = 16
NTILES = NCORES * NSUB
CHUNK = 128
CPT = 80
E_PAD = NTILES * CPT * CHUNK
ZR = 624


def _ssp(z):
    return jnp.maximum(z, 0.0) + jnp.log1p(jnp.exp(-jnp.abs(z))) - LOG2


def _h_body(x_ref, w_ref, o_ref):
    o_ref[...] = jnp.dot(x_ref[...], w_ref[...], preferred_element_type=jnp.float32)


def _node_linear(x, w):
    return pl.pallas_call(
        _h_body,
        grid=(10,),
        in_specs=[
            pl.BlockSpec((N_NODES // 10, D), lambda i: (i, 0)),
            pl.BlockSpec((D, D), lambda i: (0, 0)),
        ],
        out_specs=pl.BlockSpec((N_NODES // 10, D), lambda i: (i, 0)),
        out_shape=jax.ShapeDtypeStruct((N_NODES, D), jnp.float32),
    )(x, w)


_BE = 4096


def _we_body(ee_ref, attr_ref, wfc1_ref, b_ref, wfc2_ref, o_ref):
    z = jnp.dot(ee_ref[...], wfc1_ref[...], preferred_element_type=jnp.float32)
    z = z + b_ref[...]
    t = _ssp(z) * attr_ref[...]
    o_ref[...] = jnp.dot(t, wfc2_ref[...], preferred_element_type=jnp.float32)


def _edge_weights(ee_pad, attr_pad, wfc1, bfc1, wfc2):
    return pl.pallas_call(
        _we_body,
        grid=(E_PAD // _BE,),
        in_specs=[
            pl.BlockSpec((_BE, D_EDGE), lambda i: (i, 0)),
            pl.BlockSpec((_BE, 1), lambda i: (i, 0)),
            pl.BlockSpec((D_EDGE, FC_HIDDEN), lambda i: (0, 0)),
            pl.BlockSpec((1, FC_HIDDEN), lambda i: (0, 0)),
            pl.BlockSpec((FC_HIDDEN, D), lambda i: (0, 0)),
        ],
        out_specs=pl.BlockSpec((_BE, D), lambda i: (i, 0)),
        out_shape=jax.ShapeDtypeStruct((E_PAD, D), jnp.float32),
    )(ee_pad, attr_pad, wfc1, bfc1.reshape(1, FC_HIDDEN), wfc2)


def _sc_agg_body(h_hbm, src_hbm, dst_hbm, we_hbm, out_hbm,
                 src_c, dst_c, rows_v, we_v, acc, sem):
    c = lax.axis_index("c")
    s = lax.axis_index("s")
    w = c * NSUB + s

    def _zrow(r, carry):
        for k in range(8):
            we_v[r, pl.ds(k * 16, 16)] = jnp.zeros((16,), jnp.float32)
        return carry

    lax.fori_loop(0, CHUNK, _zrow, 0)
    base_r = s * ZR
    for t in range(ZR // CHUNK):
        pltpu.sync_copy(we_v, acc.at[pl.ds(base_r + t * CHUNK, CHUNK)])
    pltpu.sync_copy(
        we_v.at[pl.ds(0, ZR % CHUNK)],
        acc.at[pl.ds(base_r + (ZR // CHUNK) * CHUNK, ZR % CHUNK)])

    @pl.when(s == NSUB - 1)
    def _zero_tail():
        pltpu.sync_copy(we_v.at[pl.ds(0, N_NODES - NSUB * ZR)],
                        acc.at[pl.ds(NSUB * ZR, N_NODES - NSUB * ZR)])

    plsc.subcore_barrier()

    def _chunk(j, carry):
        g = w * CPT + j
        pltpu.sync_copy(src_hbm.at[pl.ds(g * CHUNK, CHUNK)], src_c)
        pltpu.sync_copy(dst_hbm.at[pl.ds(g * CHUNK, CHUNK)], dst_c)
        cp = pltpu.async_copy(h_hbm.at[src_c], rows_v, sem)
        pltpu.sync_copy(we_hbm.at[pl.ds(g * CHUNK, CHUNK)], we_v)
        cp.wait()

        def _mrow(r, cc):
            for k in range(8):
                sl = pl.ds(k * 16, 16)
                rows_v[r, sl] = rows_v[r, sl] * we_v[r, sl]
            return cc

        lax.fori_loop(0, CHUNK, _mrow, 0)
        pltpu.sync_copy(rows_v, acc.at[dst_c], add=True)
        return carry

    lax.fori_loop(0, CPT, _chunk, 0)
    plsc.subcore_barrier()
    pltpu.sync_copy(acc.at[pl.ds(base_r, ZR)],
                    out_hbm.at[pl.ds(c * N_NODES + base_r, ZR)])

    @pl.when(s == NSUB - 1)
    def _copy_tail():
        tail = N_NODES - NSUB * ZR
        pltpu.sync_copy(acc.at[pl.ds(NSUB * ZR, tail)],
                        out_hbm.at[pl.ds(c * N_NODES + NSUB * ZR, tail)])


def _sc_aggregate(h, src2d, dst2d, we):
    mesh = plsc.VectorSubcoreMesh(
        core_axis_name="c", subcore_axis_name="s",
        num_cores=NCORES, num_subcores=NSUB)
    kern = functools.partial(
        pl.kernel,
        mesh=mesh,
        out_type=jax.ShapeDtypeStruct((NCORES * N_NODES, D), jnp.float32),
        scratch_types=[
            pltpu.VMEM((CHUNK,), jnp.int32),
            pltpu.VMEM((CHUNK,), jnp.int32),
            pltpu.VMEM((CHUNK, D), jnp.float32),
            pltpu.VMEM((CHUNK, D), jnp.float32),
            pltpu.VMEM_SHARED((N_NODES, D), jnp.float32),
            pltpu.SemaphoreType.DMA,
        ],
    )(_sc_agg_body)
    return kern(h, src2d, dst2d, we)


def _out_body(p0_ref, p1_ref, x_ref, w2_ref, wsc_ref, o_ref):
    agg = p0_ref[...] + p1_ref[...]
    acc = jnp.dot(agg, w2_ref[...], preferred_element_type=jnp.float32)
    acc = acc + jnp.dot(x_ref[...], wsc_ref[...], preferred_element_type=jnp.float32)
    o_ref[...] = x_ref[...] + _ssp(acc)


def _final(partials, x, w2s, wsc):
    nb = N_NODES // 10
    return pl.pallas_call(
        _out_body,
        grid=(10,),
        in_specs=[
            pl.BlockSpec((nb, D), lambda i: (i, 0)),
            pl.BlockSpec((nb, D), lambda i: (i, 0)),
            pl.BlockSpec((nb, D), lambda i: (i, 0)),
            pl.BlockSpec((D, D), lambda i: (0, 0)),
            pl.BlockSpec((D, D), lambda i: (0, 0)),
        ],
        out_specs=pl.BlockSpec((nb, D), lambda i: (i, 0)),
        out_shape=jax.ShapeDtypeStruct((N_NODES, D), jnp.float32),
    )(partials[:N_NODES], partials[N_NODES:], x, w2s, wsc)


def kernel(node_features, edge_index, edge_attrs, edge_embedding,
           W1, Wfc1, bfc1, Wfc2, W2, Wsc):
    src = edge_index[0].astype(jnp.int32)
    dst = edge_index[1].astype(jnp.int32)
    pad = E_PAD - E
    src_pad = jnp.concatenate([src, jnp.zeros((pad,), jnp.int32)])
    dst_pad = jnp.concatenate([dst, jnp.zeros((pad,), jnp.int32)])
    ee_pad = jnp.concatenate(
        [edge_embedding, jnp.zeros((pad, D_EDGE), jnp.float32)])
    attr_pad = jnp.concatenate([edge_attrs, jnp.zeros((pad, 1), jnp.float32)])

    h = _node_linear(node_features, W1)
    we = _edge_weights(ee_pad, attr_pad, Wfc1, bfc1, Wfc2)
    partials = _sc_aggregate(h, src_pad, dst_pad, we)
    w2s = W2 * np.float32(1.0 / np.sqrt(AVG_NUM_NEIGHBORS))
    return _final(partials, node_features, w2s, Wsc)


# speedup vs baseline: 1.8984x; 1.1829x over previous
import functools

import jax
import jax.numpy as jnp
import numpy as np
from jax import lax
from jax.experimental import pallas as pl
from jax.experimental.pallas import tpu as pltpu
from jax.experimental.pallas import tpu_sc as plsc

N_NODES = 10000
D = 128
E = 320000
D_EDGE = 16
FC_HIDDEN = 64
AVG_NUM_NEIGHBORS = 32.0
LOG2 = float(np.log(2.0))

NCORES = 2
NSUB = 16
NTILES = NCORES * NSUB
CHUNK = 64
CPT = 160
E_PAD = NTILES * CPT * CHUNK
ZR = 624


def _ssp(z):
    return jnp.maximum(z, 0.0) + jnp.log1p(jnp.exp(-jnp.abs(z))) - LOG2


def _h_body(x_ref, w_ref, o_ref):
    o_ref[...] = jnp.dot(x_ref[...], w_ref[...], preferred_element_type=jnp.float32)


def _node_linear(x, w):
    return pl.pallas_call(
        _h_body,
        grid=(10,),
        in_specs=[
            pl.BlockSpec((N_NODES // 10, D), lambda i: (i, 0)),
            pl.BlockSpec((D, D), lambda i: (0, 0)),
        ],
        out_specs=pl.BlockSpec((N_NODES // 10, D), lambda i: (i, 0)),
        out_shape=jax.ShapeDtypeStruct((N_NODES, D), jnp.float32),
    )(x, w)


_BE = 4096


def _we_body(ee_ref, attr_ref, wfc1_ref, b_ref, wfc2_ref, o_ref):
    z = jnp.dot(ee_ref[...], wfc1_ref[...], preferred_element_type=jnp.float32)
    z = z + b_ref[...]
    t = _ssp(z) * attr_ref[...]
    o_ref[...] = jnp.dot(t, wfc2_ref[...], preferred_element_type=jnp.float32)


def _edge_weights(ee_pad, attr_pad, wfc1, bfc1, wfc2):
    return pl.pallas_call(
        _we_body,
        grid=(E_PAD // _BE,),
        in_specs=[
            pl.BlockSpec((_BE, D_EDGE), lambda i: (i, 0)),
            pl.BlockSpec((_BE, 1), lambda i: (i, 0)),
            pl.BlockSpec((D_EDGE, FC_HIDDEN), lambda i: (0, 0)),
            pl.BlockSpec((1, FC_HIDDEN), lambda i: (0, 0)),
            pl.BlockSpec((FC_HIDDEN, D), lambda i: (0, 0)),
        ],
        out_specs=pl.BlockSpec((_BE, D), lambda i: (i, 0)),
        out_shape=jax.ShapeDtypeStruct((E_PAD, D), jnp.float32),
    )(ee_pad, attr_pad, wfc1, bfc1.reshape(1, FC_HIDDEN), wfc2)


def _sc_agg_body(h_hbm, src_hbm, dst_hbm, we_hbm, out_hbm,
                 src0, src1, dst0, dst1, dst2, dst3,
                 rows0, rows1, we0, we1, acc,
                 gsem0, gsem1, wsem0, wsem1, isem0, isem1, ssem0, ssem1):
    c = lax.axis_index("c")
    s = lax.axis_index("s")
    w = c * NSUB + s
    tc0 = w * CPT
    srcb = (src0, src1)
    dstb = (dst0, dst1, dst2, dst3)
    rows = (rows0, rows1)
    web = (we0, we1)
    gsem = (gsem0, gsem1)
    wsem = (wsem0, wsem1)
    isem = (isem0, isem1)
    ssem = (ssem0, ssem1)

    def _issue_idx(g, sb, db, sem):
        pltpu.async_copy(src_hbm.at[pl.ds(g * CHUNK, CHUNK)], sb, sem)
        pltpu.async_copy(dst_hbm.at[pl.ds(g * CHUNK, CHUNK)], db, sem)

    def _wait_idx(sb, db, sem):
        pltpu.make_async_copy(src_hbm.at[pl.ds(0, CHUNK)], sb, sem).wait()
        pltpu.make_async_copy(dst_hbm.at[pl.ds(0, CHUNK)], db, sem).wait()

    _issue_idx(tc0, srcb[0], dstb[0], isem[0])
    _issue_idx(tc0 + 1, srcb[1], dstb[1], isem[1])

    def _zrow(r, carry):
        for k in range(8):
            we0[r, pl.ds(k * 16, 16)] = jnp.zeros((16,), jnp.float32)
        return carry

    lax.fori_loop(0, CHUNK, _zrow, 0)
    base_r = s * ZR
    for t in range(ZR // CHUNK):
        pltpu.sync_copy(we0, acc.at[pl.ds(base_r + t * CHUNK, CHUNK)])
    pltpu.sync_copy(
        we0.at[pl.ds(0, ZR % CHUNK)],
        acc.at[pl.ds(base_r + (ZR // CHUNK) * CHUNK, ZR % CHUNK)])

    @pl.when(s == NSUB - 1)
    def _zero_tail():
        pltpu.sync_copy(we0.at[pl.ds(0, N_NODES - NSUB * ZR)],
                        acc.at[pl.ds(NSUB * ZR, N_NODES - NSUB * ZR)])

    plsc.subcore_barrier()

    _wait_idx(srcb[0], dstb[0], isem[0])
    pltpu.async_copy(h_hbm.at[srcb[0]], rows[0], gsem[0])
    pltpu.async_copy(we_hbm.at[pl.ds(tc0 * CHUNK, CHUNK)], web[0], wsem[0])

    def _mul(rb, wb):
        def _mrow(r, cc):
            for k in range(8):
                sl = pl.ds(k * 16, 16)
                rb[r, sl] = rb[r, sl] * wb[r, sl]
            return cc
        lax.fori_loop(0, CHUNK, _mrow, 0)

    def _quad(p, carry):
        for u in range(4):
            b = u % 2
            nb = 1 - b
            jj = 4 * p + u
            g = tc0 + jj

            @pl.when(jj + 1 < CPT)
            def _start_next():
                _wait_idx(srcb[nb], dstb[(u + 1) % 4], isem[nb])

                @pl.when(jj >= 1)
                def _free_rows():
                    pltpu.make_async_copy(
                        rows[nb], acc.at[dstb[(u - 1) % 4]], ssem[nb]).wait()

                pltpu.async_copy(h_hbm.at[srcb[nb]], rows[nb], gsem[nb])
                pltpu.async_copy(
                    we_hbm.at[pl.ds((g + 1) * CHUNK, CHUNK)], web[nb], wsem[nb])

            pltpu.make_async_copy(h_hbm.at[srcb[b]], rows[b], gsem[b]).wait()
            pltpu.make_async_copy(
                we_hbm.at[pl.ds(0, CHUNK)], web[b], wsem[b]).wait()

            @pl.when(jj + 2 < CPT)
            def _fetch_idx():
                _issue_idx(g + 2, srcb[b], dstb[(u + 2) % 4], isem[b])

            _mul(rows[b], web[b])
            pltpu.async_copy(rows[b], acc.at[dstb[u]], ssem[b], add=True)
        return carry

    lax.fori_loop(0, CPT // 4, _quad, 0)

    pltpu.make_async_copy(rows[0], acc.at[dstb[2]], ssem[0]).wait()
    pltpu.make_async_copy(rows[1], acc.at[dstb[3]], ssem[1]).wait()

    plsc.subcore_barrier()
    pltpu.sync_copy(acc.at[pl.ds(base_r, ZR)],
                    out_hbm.at[pl.ds(c * N_NODES + base_r, ZR)])

    @pl.when(s == NSUB - 1)
    def _copy_tail():
        tail = N_NODES - NSUB * ZR
        pltpu.sync_copy(acc.at[pl.ds(NSUB * ZR, tail)],
                        out_hbm.at[pl.ds(c * N_NODES + NSUB * ZR, tail)])


def _sc_aggregate(h, src_pad, dst_pad, we):
    mesh = plsc.VectorSubcoreMesh(
        core_axis_name="c", subcore_axis_name="s",
        num_cores=NCORES, num_subcores=NSUB)
    kern = functools.partial(
        pl.kernel,
        mesh=mesh,
        out_type=jax.ShapeDtypeStruct((NCORES * N_NODES, D), jnp.float32),
        scratch_types=(
            [pltpu.VMEM((CHUNK,), jnp.int32) for _ in range(6)]
            + [pltpu.VMEM((CHUNK, D), jnp.float32) for _ in range(4)]
            + [pltpu.VMEM_SHARED((N_NODES, D), jnp.float32)]
            + [pltpu.SemaphoreType.DMA for _ in range(8)]
        ),
    )(_sc_agg_body)
    return kern(h, src_pad, dst_pad, we)


def _out_body(p0_ref, p1_ref, x_ref, w2_ref, wsc_ref, o_ref):
    agg = p0_ref[...] + p1_ref[...]
    acc = jnp.dot(agg, w2_ref[...], preferred_element_type=jnp.float32)
    acc = acc + jnp.dot(x_ref[...], wsc_ref[...], preferred_element_type=jnp.float32)
    o_ref[...] = x_ref[...] + _ssp(acc)


def _final(partials, x, w2s, wsc):
    nb = N_NODES // 10
    return pl.pallas_call(
        _out_body,
        grid=(10,),
        in_specs=[
            pl.BlockSpec((nb, D), lambda i: (i, 0)),
            pl.BlockSpec((nb, D), lambda i: (i, 0)),
            pl.BlockSpec((nb, D), lambda i: (i, 0)),
            pl.BlockSpec((D, D), lambda i: (0, 0)),
            pl.BlockSpec((D, D), lambda i: (0, 0)),
        ],
        out_specs=pl.BlockSpec((nb, D), lambda i: (i, 0)),
        out_shape=jax.ShapeDtypeStruct((N_NODES, D), jnp.float32),
    )(partials[:N_NODES], partials[N_NODES:], x, w2s, wsc)


def kernel(node_features, edge_index, edge_attrs, edge_embedding,
           W1, Wfc1, bfc1, Wfc2, W2, Wsc):
    src = edge_index[0].astype(jnp.int32)
    dst = edge_index[1].astype(jnp.int32)
    pad = E_PAD - E
    src_pad = jnp.concatenate([src, jnp.zeros((pad,), jnp.int32)])
    dst_pad = jnp.concatenate([dst, jnp.zeros((pad,), jnp.int32)])
    ee_pad = jnp.concatenate(
        [edge_embedding, jnp.zeros((pad, D_EDGE), jnp.float32)])
    attr_pad = jnp.concatenate([edge_attrs, jnp.zeros((pad, 1), jnp.float32)])

    h = _node_linear(node_features, W1)
    we = _edge_weights(ee_pad, attr_pad, Wfc1, bfc1, Wfc2)
    partials = _sc_aggregate(h, src_pad, dst_pad, we)
    w2s = W2 * np.float32(1.0 / np.sqrt(AVG_NUM_NEIGHBORS))
    return _final(partials, node_features, w2s, Wsc)
